# Initial kernel scaffold; baseline (speedup 1.0000x reference)
import jax
import jax.numpy as jnp
from jax.experimental import pallas as pl


def kernel(xyz, Wf1, bf1, Wf2, bf2, Wc1, bc1, Wc2, bc2):
    raise NotImplementedError("write your pallas kernel here")


# speedup vs baseline: 2.1154x; 2.1154x over previous
import jax
import jax.numpy as jnp
from jax.experimental import pallas as pl
from jax.experimental.pallas import tpu as pltpu

_N_VIEW = 4
_N_POINT = 1024
_T = 0.01
_PAIRS_I = (0, 0, 0, 1, 1, 2)
_PAIRS_J = (1, 2, 3, 2, 3, 3)


def _pair_kernel(pi_ref, pjt_ref,
                 wf1_ref, bf1r_ref, wf2_ref, bf2r_ref,
                 wc1_ref, bc1r_ref, wc2_ref, bc2r_ref,
                 wf1t_ref, bf1c_ref, wf2t_ref, bf2c_ref,
                 wc1t_ref, bc1c_ref, wc2t_ref,
                 out_ref):
    f32 = jnp.float32
    pc_i = pi_ref[0]
    pc_jt = pjt_ref[0]

    h_i = jnp.tanh(jnp.dot(pc_i, wf1_ref[...], preferred_element_type=f32)
                   + bf1r_ref[...])
    a_i = pc_i + jnp.dot(h_i, wf2_ref[...], preferred_element_type=f32) \
        + bf2r_ref[...]
    hc_i = jnp.tanh(jnp.dot(a_i, wc1_ref[...], preferred_element_type=f32)
                    + bc1r_ref[...])
    w_i = jax.nn.sigmoid(
        jnp.dot(hc_i, wc2_ref[...], preferred_element_type=f32)
        + bc2r_ref[...])

    h_jt = jnp.tanh(jnp.dot(wf1t_ref[...], pc_jt, preferred_element_type=f32)
                    + bf1c_ref[...])
    b_jt = pc_jt + jnp.dot(wf2t_ref[...], h_jt, preferred_element_type=f32) \
        + bf2c_ref[...]
    hc_jt = jnp.tanh(jnp.dot(wc1t_ref[...], b_jt, preferred_element_type=f32)
                     + bc1c_ref[...])
    w_j = jax.nn.sigmoid(
        jnp.dot(wc2t_ref[...], hc_jt, preferred_element_type=f32)
        + bc2r_ref[...])

    d2_12 = jnp.zeros((_N_POINT, _N_POINT), f32)
    d2_21 = jnp.zeros((_N_POINT, _N_POINT), f32)
    for k in range(3):
        diff12 = a_i[:, k:k + 1] - pc_jt[k:k + 1, :]
        d2_12 = d2_12 + diff12 * diff12
        diff21 = pc_i[:, k:k + 1] - b_jt[k:k + 1, :]
        d2_21 = d2_21 + diff21 * diff21
    d12 = jnp.sqrt(jnp.maximum(d2_12, 1e-12))
    d21 = jnp.sqrt(jnp.maximum(d2_21, 1e-12))

    s = w_i + w_j
    logits = -(d12 * w_i + d21 * w_j) / (s * _T)
    m = jnp.max(logits, axis=1, keepdims=True)
    e = jnp.exp(logits - m)
    out_ref[0] = e / jnp.sum(e, axis=1, keepdims=True)


def kernel(xyz, Wf1, bf1, Wf2, bf2, Wc1, bc1, Wc2, bc2):
    x = xyz[0]
    pi = jnp.stack([x[i] for i in _PAIRS_I])
    pjt = jnp.stack([x[j].T for j in _PAIRS_J])

    full = lambda shape: pl.BlockSpec(shape, lambda p: (0,) * len(shape))
    in_specs = [
        pl.BlockSpec((1, _N_POINT, 3), lambda p: (p, 0, 0)),
        pl.BlockSpec((1, 3, _N_POINT), lambda p: (p, 0, 0)),
        full((3, 64)), full((1, 64)), full((64, 3)), full((1, 3)),
        full((3, 64)), full((1, 64)), full((64, 1)), full((1, 1)),
        full((64, 3)), full((64, 1)), full((3, 64)), full((3, 1)),
        full((64, 3)), full((64, 1)), full((1, 64)),
    ]
    out = pl.pallas_call(
        _pair_kernel,
        grid=(6,),
        in_specs=in_specs,
        out_specs=pl.BlockSpec((1, _N_POINT, _N_POINT), lambda p: (p, 0, 0)),
        out_shape=jax.ShapeDtypeStruct((6, _N_POINT, _N_POINT), jnp.float32),
        compiler_params=pltpu.CompilerParams(
            dimension_semantics=("arbitrary",)),
    )(
        pi, pjt,
        Wf1, bf1.reshape(1, 64), Wf2, bf2.reshape(1, 3),
        Wc1, bc1.reshape(1, 64), Wc2, bc2.reshape(1, 1),
        Wf1.T, bf1.reshape(64, 1), Wf2.T, bf2.reshape(3, 1),
        Wc1.T, bc1.reshape(64, 1), Wc2.T,
    )
    return out.reshape(6, 1, _N_POINT, _N_POINT)
